# Initial kernel scaffold; baseline (speedup 1.0000x reference)
import jax
import jax.numpy as jnp
from jax.experimental import pallas as pl


def kernel(nfeat, efeat, edge_index, W1, b1, W2, b2, gn1_alpha, gn1_gamma, gn1_beta, gn2_alpha, gn2_gamma, gn2_beta):
    raise NotImplementedError("write your pallas kernel here")


# speedup vs baseline: 2.4440x; 2.4440x over previous
import functools

import jax
import jax.numpy as jnp
from jax import lax
from jax.experimental import pallas as pl
from jax.experimental.pallas import tpu as pltpu
from jax.experimental.pallas import tpu_sc as plsc

N_NODES = 10000
N_EDGES = 160000
D = 256
HALF = 128
NSUB = 16
EDGES_PER_SUB = N_EDGES // NSUB
CHUNK = 80
NCHUNK = EDGES_PER_SUB // CHUNK
NODE_CHUNK = 80
N_NODE_CHUNKS = N_NODES // NODE_CHUNK
NODE_ROUNDS = (N_NODE_CHUNKS + NSUB - 1) // NSUB
GN_EPS = 1e-5


def _sc_aggregate(nfeat, nfeat2, efeat, src2, dst):
    mesh = plsc.VectorSubcoreMesh(core_axis_name="core", subcore_axis_name="subcore")

    @functools.partial(
        pl.kernel,
        out_type=jax.ShapeDtypeStruct((N_NODES, D), jnp.float32),
        mesh=mesh,
        scratch_types=[
            pltpu.VMEM((CHUNK,), jnp.int32),
            pltpu.VMEM((CHUNK,), jnp.int32),
            pltpu.VMEM((CHUNK, HALF), jnp.float32),
            pltpu.VMEM((CHUNK, HALF), jnp.float32),
            pltpu.VMEM((NODE_CHUNK, HALF), jnp.float32),
            pltpu.VMEM_SHARED((N_NODES, HALF), jnp.float32),
        ],
    )
    def k(nfeat_hbm, nfeat2_hbm, efeat_hbm, src2_hbm, dst_hbm, out_hbm,
          gidx_v, didx_v, grow_v, erow_v, bounce_v, agg_sh):
        c = lax.axis_index("core")
        s = lax.axis_index("subcore")
        f0 = c * HALF

        @pl.loop(0, NODE_ROUNDS)
        def _(kk):
            cid = s + kk * NSUB

            @pl.when(cid < N_NODE_CHUNKS)
            def _():
                n0 = cid * NODE_CHUNK
                pltpu.sync_copy(
                    nfeat_hbm.at[pl.ds(n0, NODE_CHUNK), pl.ds(f0, HALF)], bounce_v)
                pltpu.sync_copy(bounce_v, agg_sh.at[pl.ds(n0, NODE_CHUNK)])

        plsc.subcore_barrier()

        @pl.loop(0, NCHUNK)
        def _(j):
            e0 = s * EDGES_PER_SUB + j * CHUNK
            pltpu.sync_copy(src2_hbm.at[pl.ds(e0, CHUNK)], gidx_v)
            pltpu.sync_copy(dst_hbm.at[pl.ds(e0, CHUNK)], didx_v)
            for i in range(CHUNK // 16):
                gidx_v[pl.ds(16 * i, 16)] = gidx_v[pl.ds(16 * i, 16)] + c
            pltpu.sync_copy(nfeat2_hbm.at[gidx_v], grow_v)
            pltpu.sync_copy(efeat_hbm.at[pl.ds(e0, CHUNK), pl.ds(f0, HALF)], erow_v)
            pltpu.sync_copy(grow_v, agg_sh.at[didx_v], add=True)
            pltpu.sync_copy(erow_v, agg_sh.at[didx_v], add=True)

        plsc.subcore_barrier()

        @pl.loop(0, NODE_ROUNDS)
        def _(kk):
            cid = s + kk * NSUB

            @pl.when(cid < N_NODE_CHUNKS)
            def _():
                n0 = cid * NODE_CHUNK
                pltpu.sync_copy(agg_sh.at[pl.ds(n0, NODE_CHUNK)], bounce_v)
                pltpu.sync_copy(
                    bounce_v, out_hbm.at[pl.ds(n0, NODE_CHUNK), pl.ds(f0, HALF)])

    return k(nfeat, nfeat2, efeat, src2, dst)


BR = 2000
NB = N_NODES // BR


def _mm1_body(h_ref, w1_ref, b1_ref, rst1_ref, st1_ref):
    i = pl.program_id(0)
    y = jnp.dot(h_ref[...], w1_ref[...], preferred_element_type=jnp.float32)
    y = y + b1_ref[...]
    rst1_ref[...] = y

    @pl.when(i == 0)
    def _():
        st1_ref[...] = jnp.zeros_like(st1_ref)

    st1_ref[0:1, :] += jnp.sum(y, axis=0, keepdims=True)
    st1_ref[1:2, :] += jnp.sum(y * y, axis=0, keepdims=True)


def _norm_from_stats(st_ref, a):
    mean = st_ref[0:1, :] * (1.0 / N_NODES)
    msq = st_ref[1:2, :] * (1.0 / N_NODES)
    var = msq - (2.0 * a - a * a) * mean * mean
    rstd = lax.rsqrt(var + GN_EPS)
    return mean, rstd


def _mm2_body(rst1_ref, st1_ref, a1_ref, g1_ref, be1_ref, w2_ref, b2_ref,
              rst2_ref, st2_ref):
    i = pl.program_id(0)
    a = a1_ref[...]
    mean, rstd = _norm_from_stats(st1_ref, a)
    xn = g1_ref[...] * ((rst1_ref[...] - a * mean) * rstd) + be1_ref[...]
    r = jnp.maximum(xn, 0.0)
    y = jnp.dot(r, w2_ref[...], preferred_element_type=jnp.float32)
    y = y + b2_ref[...]
    rst2_ref[...] = y

    @pl.when(i == 0)
    def _():
        st2_ref[...] = jnp.zeros_like(st2_ref)

    st2_ref[0:1, :] += jnp.sum(y, axis=0, keepdims=True)
    st2_ref[1:2, :] += jnp.sum(y * y, axis=0, keepdims=True)


def _norm_body(rst2_ref, st2_ref, a2_ref, g2_ref, be2_ref, out_ref):
    a = a2_ref[...]
    mean, rstd = _norm_from_stats(st2_ref, a)
    out_ref[...] = g2_ref[...] * ((rst2_ref[...] - a * mean) * rstd) + be2_ref[...]


def _row(v):
    return v.reshape(1, -1)


def _mlp(h, W1, b1, W2, b2, gn1_alpha, gn1_gamma, gn1_beta,
         gn2_alpha, gn2_gamma, gn2_beta):
    D2 = 2 * D
    rst1, st1 = pl.pallas_call(
        _mm1_body,
        grid=(NB,),
        in_specs=[
            pl.BlockSpec((BR, D), lambda i: (i, 0)),
            pl.BlockSpec((D, D2), lambda i: (0, 0)),
            pl.BlockSpec((1, D2), lambda i: (0, 0)),
        ],
        out_specs=[
            pl.BlockSpec((BR, D2), lambda i: (i, 0)),
            pl.BlockSpec((8, D2), lambda i: (0, 0)),
        ],
        out_shape=[
            jax.ShapeDtypeStruct((N_NODES, D2), jnp.float32),
            jax.ShapeDtypeStruct((8, D2), jnp.float32),
        ],
    )(h, W1, _row(b1))

    rst2, st2 = pl.pallas_call(
        _mm2_body,
        grid=(NB,),
        in_specs=[
            pl.BlockSpec((BR, D2), lambda i: (i, 0)),
            pl.BlockSpec((8, D2), lambda i: (0, 0)),
            pl.BlockSpec((1, D2), lambda i: (0, 0)),
            pl.BlockSpec((1, D2), lambda i: (0, 0)),
            pl.BlockSpec((1, D2), lambda i: (0, 0)),
            pl.BlockSpec((D2, D), lambda i: (0, 0)),
            pl.BlockSpec((1, D), lambda i: (0, 0)),
        ],
        out_specs=[
            pl.BlockSpec((BR, D), lambda i: (i, 0)),
            pl.BlockSpec((8, D), lambda i: (0, 0)),
        ],
        out_shape=[
            jax.ShapeDtypeStruct((N_NODES, D), jnp.float32),
            jax.ShapeDtypeStruct((8, D), jnp.float32),
        ],
    )(rst1, st1, _row(gn1_alpha), _row(gn1_gamma), _row(gn1_beta), W2, _row(b2))

    out = pl.pallas_call(
        _norm_body,
        grid=(NB,),
        in_specs=[
            pl.BlockSpec((BR, D), lambda i: (i, 0)),
            pl.BlockSpec((8, D), lambda i: (0, 0)),
            pl.BlockSpec((1, D), lambda i: (0, 0)),
            pl.BlockSpec((1, D), lambda i: (0, 0)),
            pl.BlockSpec((1, D), lambda i: (0, 0)),
        ],
        out_specs=pl.BlockSpec((BR, D), lambda i: (i, 0)),
        out_shape=jax.ShapeDtypeStruct((N_NODES, D), jnp.float32),
    )(rst2, st2, _row(gn2_alpha), _row(gn2_gamma), _row(gn2_beta))
    return out


@jax.jit
def kernel(nfeat, efeat, edge_index, W1, b1, W2, b2,
           gn1_alpha, gn1_gamma, gn1_beta, gn2_alpha, gn2_gamma, gn2_beta):
    src = edge_index[0].astype(jnp.int32)
    dst = edge_index[1].astype(jnp.int32)
    src2 = src * 2
    nfeat2 = nfeat.reshape(2 * N_NODES, HALF)
    h = _sc_aggregate(nfeat, nfeat2, efeat, src2, dst)
    return _mlp(h, W1, b1, W2, b2, gn1_alpha, gn1_gamma, gn1_beta,
                gn2_alpha, gn2_gamma, gn2_beta)


# speedup vs baseline: 4.7055x; 1.9254x over previous
import functools

import jax
import jax.numpy as jnp
from jax import lax
from jax.experimental import pallas as pl
from jax.experimental.pallas import tpu as pltpu
from jax.experimental.pallas import tpu_sc as plsc

N_NODES = 10000
N_EDGES = 160000
D = 256
HALF = 128
NSUB = 16
EDGES_PER_SUB = N_EDGES // NSUB
CHUNK = 80
NCHUNK = EDGES_PER_SUB // CHUNK
NODE_CHUNK = 80
N_NODE_CHUNKS = N_NODES // NODE_CHUNK
NODE_ROUNDS = (N_NODE_CHUNKS + NSUB - 1) // NSUB
GN_EPS = 1e-5


def _sc_aggregate(nfeat, nfeat2, efeat, src2, dst):
    mesh = plsc.VectorSubcoreMesh(core_axis_name="core", subcore_axis_name="subcore")

    @functools.partial(
        pl.kernel,
        out_type=jax.ShapeDtypeStruct((N_NODES, D), jnp.float32),
        mesh=mesh,
        scratch_types=[
            pltpu.VMEM((2, CHUNK), jnp.int32),
            pltpu.VMEM((3, CHUNK), jnp.int32),
            pltpu.VMEM((2, CHUNK, HALF), jnp.float32),
            pltpu.VMEM((2, CHUNK, HALF), jnp.float32),
            pltpu.VMEM_SHARED((N_NODES, HALF), jnp.float32),
            pltpu.SemaphoreType.DMA((2,)),
            pltpu.SemaphoreType.DMA((2,)),
            pltpu.SemaphoreType.DMA((2,)),
            pltpu.SemaphoreType.DMA((2,)),
            pltpu.SemaphoreType.DMA((2,)),
            pltpu.SemaphoreType.DMA((2,)),
        ],
    )
    def k(nfeat_hbm, nfeat2_hbm, efeat_hbm, src2_hbm, dst_hbm, out_hbm,
          gidx_v, didx_v, grow_v, erow_v, agg_sh,
          semI, semD, semGn, semGe, semSg, semSe):
        c = lax.axis_index("core")
        s = lax.axis_index("subcore")
        f0 = c * HALF

        def e_window(e0):
            return efeat_hbm.at[pl.ds(e0, CHUNK), pl.ds(f0, HALF)]

        def start_idx(j, p, q):
            e0 = s * EDGES_PER_SUB + j * CHUNK
            pltpu.async_copy(src2_hbm.at[pl.ds(e0, CHUNK)], gidx_v.at[p], semI.at[p])
            pltpu.async_copy(dst_hbm.at[pl.ds(e0, CHUNK)], didx_v.at[q], semD.at[p])

        def wait_idx(j, p, q):
            e0 = s * EDGES_PER_SUB + j * CHUNK
            pltpu.make_async_copy(
                src2_hbm.at[pl.ds(e0, CHUNK)], gidx_v.at[p], semI.at[p]).wait()
            pltpu.make_async_copy(
                dst_hbm.at[pl.ds(e0, CHUNK)], didx_v.at[q], semD.at[p]).wait()

        def start_gather(j, p):
            e0 = s * EDGES_PER_SUB + j * CHUNK
            pltpu.async_copy(nfeat2_hbm.at[gidx_v.at[p]], grow_v.at[p], semGn.at[p])
            pltpu.async_copy(e_window(e0), erow_v.at[p], semGe.at[p])

        def wait_gather(p):
            pltpu.make_async_copy(
                nfeat2_hbm.at[gidx_v.at[p]], grow_v.at[p], semGn.at[p]).wait()
            pltpu.make_async_copy(e_window(0), erow_v.at[p], semGe.at[p]).wait()

        def start_scatter(p, q):
            pltpu.async_copy(grow_v.at[p], agg_sh.at[didx_v.at[q]], semSg.at[p],
                             add=True)
            pltpu.async_copy(erow_v.at[p], agg_sh.at[didx_v.at[q]], semSe.at[p],
                             add=True)

        def wait_scatter(p, q):
            pltpu.make_async_copy(
                grow_v.at[p], agg_sh.at[didx_v.at[q]], semSg.at[p]).wait()
            pltpu.make_async_copy(
                erow_v.at[p], agg_sh.at[didx_v.at[q]], semSe.at[p]).wait()

        @pl.loop(0, NODE_ROUNDS)
        def _(kk):
            cid = s + kk * NSUB

            @pl.when(cid < N_NODE_CHUNKS)
            def _():
                n0 = cid * NODE_CHUNK
                pltpu.sync_copy(
                    nfeat_hbm.at[pl.ds(n0, NODE_CHUNK), pl.ds(f0, HALF)],
                    grow_v.at[0])
                pltpu.sync_copy(grow_v.at[0], agg_sh.at[pl.ds(n0, NODE_CHUNK)])

        plsc.subcore_barrier()

        start_idx(0, 0, 0)

        @pl.loop(0, NCHUNK)
        def _(j):
            p = lax.rem(j, 2)
            pn = 1 - p
            q = lax.rem(j, 3)

            @pl.when(j >= 1)
            def _():
                qm = lax.rem(j + 2, 3)
                wait_gather(pn)
                start_scatter(pn, qm)

            @pl.when(j >= 2)
            def _():
                wait_scatter(p, lax.rem(j + 1, 3))

            @pl.when(j + 1 < NCHUNK)
            def _():
                start_idx(j + 1, pn, lax.rem(j + 1, 3))

            wait_idx(j, p, q)
            for i in range(CHUNK // 16):
                gidx_v[p, pl.ds(16 * i, 16)] = gidx_v[p, pl.ds(16 * i, 16)] + c
            start_gather(j, p)

        pl_ = (NCHUNK - 1) % 2
        ql_ = (NCHUNK - 1) % 3
        wait_gather(pl_)
        start_scatter(pl_, ql_)
        wait_scatter(1 - pl_, (NCHUNK - 2) % 3)
        wait_scatter(pl_, ql_)

        plsc.subcore_barrier()

        @pl.loop(0, NODE_ROUNDS)
        def _(kk):
            cid = s + kk * NSUB

            @pl.when(cid < N_NODE_CHUNKS)
            def _():
                n0 = cid * NODE_CHUNK
                pltpu.sync_copy(agg_sh.at[pl.ds(n0, NODE_CHUNK)], grow_v.at[0])
                pltpu.sync_copy(
                    grow_v.at[0], out_hbm.at[pl.ds(n0, NODE_CHUNK), pl.ds(f0, HALF)])

    return k(nfeat, nfeat2, efeat, src2, dst)


BR = 2000
NB = N_NODES // BR


def _mm1_body(h_ref, w1_ref, b1_ref, rst1_ref, st1_ref):
    i = pl.program_id(0)
    y = jnp.dot(h_ref[...], w1_ref[...], preferred_element_type=jnp.float32)
    y = y + b1_ref[...]
    rst1_ref[...] = y

    @pl.when(i == 0)
    def _():
        st1_ref[...] = jnp.zeros_like(st1_ref)

    st1_ref[0:1, :] += jnp.sum(y, axis=0, keepdims=True)
    st1_ref[1:2, :] += jnp.sum(y * y, axis=0, keepdims=True)


def _norm_from_stats(st_ref, a):
    mean = st_ref[0:1, :] * (1.0 / N_NODES)
    msq = st_ref[1:2, :] * (1.0 / N_NODES)
    var = msq - (2.0 * a - a * a) * mean * mean
    rstd = lax.rsqrt(var + GN_EPS)
    return mean, rstd


def _mm2_body(rst1_ref, st1_ref, a1_ref, g1_ref, be1_ref, w2_ref, b2_ref,
              rst2_ref, st2_ref):
    i = pl.program_id(0)
    a = a1_ref[...]
    mean, rstd = _norm_from_stats(st1_ref, a)
    xn = g1_ref[...] * ((rst1_ref[...] - a * mean) * rstd) + be1_ref[...]
    r = jnp.maximum(xn, 0.0)
    y = jnp.dot(r, w2_ref[...], preferred_element_type=jnp.float32)
    y = y + b2_ref[...]
    rst2_ref[...] = y

    @pl.when(i == 0)
    def _():
        st2_ref[...] = jnp.zeros_like(st2_ref)

    st2_ref[0:1, :] += jnp.sum(y, axis=0, keepdims=True)
    st2_ref[1:2, :] += jnp.sum(y * y, axis=0, keepdims=True)


def _norm_body(rst2_ref, st2_ref, a2_ref, g2_ref, be2_ref, out_ref):
    a = a2_ref[...]
    mean, rstd = _norm_from_stats(st2_ref, a)
    out_ref[...] = g2_ref[...] * ((rst2_ref[...] - a * mean) * rstd) + be2_ref[...]


def _row(v):
    return v.reshape(1, -1)


def _mlp(h, W1, b1, W2, b2, gn1_alpha, gn1_gamma, gn1_beta,
         gn2_alpha, gn2_gamma, gn2_beta):
    D2 = 2 * D
    rst1, st1 = pl.pallas_call(
        _mm1_body,
        grid=(NB,),
        in_specs=[
            pl.BlockSpec((BR, D), lambda i: (i, 0)),
            pl.BlockSpec((D, D2), lambda i: (0, 0)),
            pl.BlockSpec((1, D2), lambda i: (0, 0)),
        ],
        out_specs=[
            pl.BlockSpec((BR, D2), lambda i: (i, 0)),
            pl.BlockSpec((8, D2), lambda i: (0, 0)),
        ],
        out_shape=[
            jax.ShapeDtypeStruct((N_NODES, D2), jnp.float32),
            jax.ShapeDtypeStruct((8, D2), jnp.float32),
        ],
    )(h, W1, _row(b1))

    rst2, st2 = pl.pallas_call(
        _mm2_body,
        grid=(NB,),
        in_specs=[
            pl.BlockSpec((BR, D2), lambda i: (i, 0)),
            pl.BlockSpec((8, D2), lambda i: (0, 0)),
            pl.BlockSpec((1, D2), lambda i: (0, 0)),
            pl.BlockSpec((1, D2), lambda i: (0, 0)),
            pl.BlockSpec((1, D2), lambda i: (0, 0)),
            pl.BlockSpec((D2, D), lambda i: (0, 0)),
            pl.BlockSpec((1, D), lambda i: (0, 0)),
        ],
        out_specs=[
            pl.BlockSpec((BR, D), lambda i: (i, 0)),
            pl.BlockSpec((8, D), lambda i: (0, 0)),
        ],
        out_shape=[
            jax.ShapeDtypeStruct((N_NODES, D), jnp.float32),
            jax.ShapeDtypeStruct((8, D), jnp.float32),
        ],
    )(rst1, st1, _row(gn1_alpha), _row(gn1_gamma), _row(gn1_beta), W2, _row(b2))

    out = pl.pallas_call(
        _norm_body,
        grid=(NB,),
        in_specs=[
            pl.BlockSpec((BR, D), lambda i: (i, 0)),
            pl.BlockSpec((8, D), lambda i: (0, 0)),
            pl.BlockSpec((1, D), lambda i: (0, 0)),
            pl.BlockSpec((1, D), lambda i: (0, 0)),
            pl.BlockSpec((1, D), lambda i: (0, 0)),
        ],
        out_specs=pl.BlockSpec((BR, D), lambda i: (i, 0)),
        out_shape=jax.ShapeDtypeStruct((N_NODES, D), jnp.float32),
    )(rst2, st2, _row(gn2_alpha), _row(gn2_gamma), _row(gn2_beta))
    return out


@jax.jit
def kernel(nfeat, efeat, edge_index, W1, b1, W2, b2,
           gn1_alpha, gn1_gamma, gn1_beta, gn2_alpha, gn2_gamma, gn2_beta):
    src = edge_index[0].astype(jnp.int32)
    dst = edge_index[1].astype(jnp.int32)
    src2 = src * 2
    nfeat2 = nfeat.reshape(2 * N_NODES, HALF)
    h = _sc_aggregate(nfeat, nfeat2, efeat, src2, dst)
    return _mlp(h, W1, b1, W2, b2, gn1_alpha, gn1_gamma, gn1_beta,
                gn2_alpha, gn2_gamma, gn2_beta)


# speedup vs baseline: 4.8046x; 1.0210x over previous
import functools

import jax
import jax.numpy as jnp
from jax import lax
from jax.experimental import pallas as pl
from jax.experimental.pallas import tpu as pltpu
from jax.experimental.pallas import tpu_sc as plsc

N_NODES = 10000
N_EDGES = 160000
D = 256
HALF = 128
NSUB = 16
EDGES_PER_SUB = N_EDGES // NSUB
CHUNK = 80
NCHUNK = EDGES_PER_SUB // CHUNK
NODE_CHUNK = 80
N_NODE_CHUNKS = N_NODES // NODE_CHUNK
NODE_ROUNDS = (N_NODE_CHUNKS + NSUB - 1) // NSUB
GN_EPS = 1e-5


def _sc_aggregate(nfeat2, efeat, src2, dst):
    mesh = plsc.VectorSubcoreMesh(core_axis_name="core", subcore_axis_name="subcore")

    @functools.partial(
        pl.kernel,
        out_type=jax.ShapeDtypeStruct((N_NODES, D), jnp.float32),
        mesh=mesh,
        scratch_types=[
            pltpu.VMEM((2, CHUNK), jnp.int32),
            pltpu.VMEM((3, CHUNK), jnp.int32),
            pltpu.VMEM((2, CHUNK, HALF), jnp.float32),
            pltpu.VMEM((2, CHUNK, HALF), jnp.float32),
            pltpu.VMEM_SHARED((N_NODES, HALF), jnp.float32),
            pltpu.SemaphoreType.DMA((2,)),
            pltpu.SemaphoreType.DMA((2,)),
            pltpu.SemaphoreType.DMA((2,)),
            pltpu.SemaphoreType.DMA((2,)),
            pltpu.SemaphoreType.DMA((2,)),
            pltpu.SemaphoreType.DMA((2,)),
        ],
    )
    def k(nfeat2_hbm, efeat_hbm, src2_hbm, dst_hbm, out_hbm,
          gidx_v, didx_v, grow_v, erow_v, agg_sh,
          semI, semD, semGn, semGe, semSg, semSe):
        c = lax.axis_index("core")
        s = lax.axis_index("subcore")
        f0 = c * HALF

        def e_window(e0):
            return efeat_hbm.at[pl.ds(e0, CHUNK), pl.ds(f0, HALF)]

        def start_idx(j, p, q):
            e0 = s * EDGES_PER_SUB + j * CHUNK
            pltpu.async_copy(src2_hbm.at[pl.ds(e0, CHUNK)], gidx_v.at[p], semI.at[p])
            pltpu.async_copy(dst_hbm.at[pl.ds(e0, CHUNK)], didx_v.at[q], semD.at[p])

        def wait_idx(j, p, q):
            e0 = s * EDGES_PER_SUB + j * CHUNK
            pltpu.make_async_copy(
                src2_hbm.at[pl.ds(e0, CHUNK)], gidx_v.at[p], semI.at[p]).wait()
            pltpu.make_async_copy(
                dst_hbm.at[pl.ds(e0, CHUNK)], didx_v.at[q], semD.at[p]).wait()

        def start_gather(j, p):
            e0 = s * EDGES_PER_SUB + j * CHUNK
            pltpu.async_copy(nfeat2_hbm.at[gidx_v.at[p]], grow_v.at[p], semGn.at[p])
            pltpu.async_copy(e_window(e0), erow_v.at[p], semGe.at[p])

        def wait_gather(p):
            pltpu.make_async_copy(
                nfeat2_hbm.at[gidx_v.at[p]], grow_v.at[p], semGn.at[p]).wait()
            pltpu.make_async_copy(e_window(0), erow_v.at[p], semGe.at[p]).wait()

        def start_scatter(p, q):
            pltpu.async_copy(grow_v.at[p], agg_sh.at[didx_v.at[q]], semSg.at[p],
                             add=True)
            pltpu.async_copy(erow_v.at[p], agg_sh.at[didx_v.at[q]], semSe.at[p],
                             add=True)

        def wait_scatter(p, q):
            pltpu.make_async_copy(
                grow_v.at[p], agg_sh.at[didx_v.at[q]], semSg.at[p]).wait()
            pltpu.make_async_copy(
                erow_v.at[p], agg_sh.at[didx_v.at[q]], semSe.at[p]).wait()

        zeros16 = jnp.zeros((16,), jnp.float32)

        @pl.loop(0, NODE_CHUNK)
        def _(r):
            for i in range(HALF // 16):
                grow_v[0, r, pl.ds(16 * i, 16)] = zeros16

        @pl.loop(0, NODE_ROUNDS)
        def _(kk):
            cid = s + kk * NSUB

            @pl.when(cid < N_NODE_CHUNKS)
            def _():
                pltpu.async_copy(grow_v.at[0],
                                 agg_sh.at[pl.ds(cid * NODE_CHUNK, NODE_CHUNK)],
                                 semSg.at[0])

        @pl.loop(0, NODE_ROUNDS)
        def _(kk):
            @pl.when(s + kk * NSUB < N_NODE_CHUNKS)
            def _():
                pltpu.make_async_copy(
                    grow_v.at[0], agg_sh.at[pl.ds(0, NODE_CHUNK)],
                    semSg.at[0]).wait()

        plsc.subcore_barrier()

        start_idx(0, 0, 0)

        @pl.loop(0, NCHUNK)
        def _(j):
            p = lax.rem(j, 2)
            pn = 1 - p
            q = lax.rem(j, 3)

            @pl.when(j >= 1)
            def _():
                qm = lax.rem(j + 2, 3)
                wait_gather(pn)
                start_scatter(pn, qm)

            @pl.when(j >= 2)
            def _():
                wait_scatter(p, lax.rem(j + 1, 3))

            @pl.when(j + 1 < NCHUNK)
            def _():
                start_idx(j + 1, pn, lax.rem(j + 1, 3))

            wait_idx(j, p, q)
            for i in range(CHUNK // 16):
                gidx_v[p, pl.ds(16 * i, 16)] = gidx_v[p, pl.ds(16 * i, 16)] + c
            start_gather(j, p)

        pl_ = (NCHUNK - 1) % 2
        ql_ = (NCHUNK - 1) % 3
        wait_gather(pl_)
        start_scatter(pl_, ql_)
        wait_scatter(1 - pl_, (NCHUNK - 2) % 3)
        wait_scatter(pl_, ql_)

        plsc.subcore_barrier()

        @pl.loop(0, NODE_ROUNDS)
        def _(kk):
            cid = s + kk * NSUB

            @pl.when(cid < N_NODE_CHUNKS)
            def _():
                n0 = cid * NODE_CHUNK
                pltpu.async_copy(
                    agg_sh.at[pl.ds(n0, NODE_CHUNK)],
                    out_hbm.at[pl.ds(n0, NODE_CHUNK), pl.ds(f0, HALF)],
                    semSe.at[0])

        @pl.loop(0, NODE_ROUNDS)
        def _(kk):
            @pl.when(s + kk * NSUB < N_NODE_CHUNKS)
            def _():
                pltpu.make_async_copy(
                    agg_sh.at[pl.ds(0, NODE_CHUNK)],
                    out_hbm.at[pl.ds(0, NODE_CHUNK), pl.ds(f0, HALF)],
                    semSe.at[0]).wait()

    return k(nfeat2, efeat, src2, dst)


BR = 2000
NB = N_NODES // BR


def _mm1_body(agg_ref, nfeat_ref, w1_ref, b1_ref, rst1_ref, st1_ref):
    i = pl.program_id(0)
    h = agg_ref[...] + nfeat_ref[...]
    y = jnp.dot(h, w1_ref[...], preferred_element_type=jnp.float32)
    y = y + b1_ref[...]
    rst1_ref[...] = y

    @pl.when(i == 0)
    def _():
        st1_ref[...] = jnp.zeros_like(st1_ref)

    st1_ref[0:1, :] += jnp.sum(y, axis=0, keepdims=True)
    st1_ref[1:2, :] += jnp.sum(y * y, axis=0, keepdims=True)


def _norm_from_stats(st_ref, a):
    mean = st_ref[0:1, :] * (1.0 / N_NODES)
    msq = st_ref[1:2, :] * (1.0 / N_NODES)
    var = msq - (2.0 * a - a * a) * mean * mean
    rstd = lax.rsqrt(var + GN_EPS)
    return mean, rstd


def _mm2_body(rst1_ref, st1_ref, a1_ref, g1_ref, be1_ref, w2_ref, b2_ref,
              rst2_ref, st2_ref):
    i = pl.program_id(0)
    a = a1_ref[...]
    mean, rstd = _norm_from_stats(st1_ref, a)
    xn = g1_ref[...] * ((rst1_ref[...] - a * mean) * rstd) + be1_ref[...]
    r = jnp.maximum(xn, 0.0)
    y = jnp.dot(r, w2_ref[...], preferred_element_type=jnp.float32)
    y = y + b2_ref[...]
    rst2_ref[...] = y

    @pl.when(i == 0)
    def _():
        st2_ref[...] = jnp.zeros_like(st2_ref)

    st2_ref[0:1, :] += jnp.sum(y, axis=0, keepdims=True)
    st2_ref[1:2, :] += jnp.sum(y * y, axis=0, keepdims=True)


def _norm_body(rst2_ref, st2_ref, a2_ref, g2_ref, be2_ref, out_ref):
    a = a2_ref[...]
    mean, rstd = _norm_from_stats(st2_ref, a)
    out_ref[...] = g2_ref[...] * ((rst2_ref[...] - a * mean) * rstd) + be2_ref[...]


def _row(v):
    return v.reshape(1, -1)


def _mlp(agg, nfeat, W1, b1, W2, b2, gn1_alpha, gn1_gamma, gn1_beta,
         gn2_alpha, gn2_gamma, gn2_beta):
    D2 = 2 * D
    rst1, st1 = pl.pallas_call(
        _mm1_body,
        grid=(NB,),
        in_specs=[
            pl.BlockSpec((BR, D), lambda i: (i, 0)),
            pl.BlockSpec((BR, D), lambda i: (i, 0)),
            pl.BlockSpec((D, D2), lambda i: (0, 0)),
            pl.BlockSpec((1, D2), lambda i: (0, 0)),
        ],
        out_specs=[
            pl.BlockSpec((BR, D2), lambda i: (i, 0)),
            pl.BlockSpec((8, D2), lambda i: (0, 0)),
        ],
        out_shape=[
            jax.ShapeDtypeStruct((N_NODES, D2), jnp.float32),
            jax.ShapeDtypeStruct((8, D2), jnp.float32),
        ],
    )(agg, nfeat, W1, _row(b1))

    rst2, st2 = pl.pallas_call(
        _mm2_body,
        grid=(NB,),
        in_specs=[
            pl.BlockSpec((BR, D2), lambda i: (i, 0)),
            pl.BlockSpec((8, D2), lambda i: (0, 0)),
            pl.BlockSpec((1, D2), lambda i: (0, 0)),
            pl.BlockSpec((1, D2), lambda i: (0, 0)),
            pl.BlockSpec((1, D2), lambda i: (0, 0)),
            pl.BlockSpec((D2, D), lambda i: (0, 0)),
            pl.BlockSpec((1, D), lambda i: (0, 0)),
        ],
        out_specs=[
            pl.BlockSpec((BR, D), lambda i: (i, 0)),
            pl.BlockSpec((8, D), lambda i: (0, 0)),
        ],
        out_shape=[
            jax.ShapeDtypeStruct((N_NODES, D), jnp.float32),
            jax.ShapeDtypeStruct((8, D), jnp.float32),
        ],
    )(rst1, st1, _row(gn1_alpha), _row(gn1_gamma), _row(gn1_beta), W2, _row(b2))

    out = pl.pallas_call(
        _norm_body,
        grid=(NB,),
        in_specs=[
            pl.BlockSpec((BR, D), lambda i: (i, 0)),
            pl.BlockSpec((8, D), lambda i: (0, 0)),
            pl.BlockSpec((1, D), lambda i: (0, 0)),
            pl.BlockSpec((1, D), lambda i: (0, 0)),
            pl.BlockSpec((1, D), lambda i: (0, 0)),
        ],
        out_specs=pl.BlockSpec((BR, D), lambda i: (i, 0)),
        out_shape=jax.ShapeDtypeStruct((N_NODES, D), jnp.float32),
    )(rst2, st2, _row(gn2_alpha), _row(gn2_gamma), _row(gn2_beta))
    return out


@jax.jit
def kernel(nfeat, efeat, edge_index, W1, b1, W2, b2,
           gn1_alpha, gn1_gamma, gn1_beta, gn2_alpha, gn2_gamma, gn2_beta):
    src = edge_index[0].astype(jnp.int32)
    dst = edge_index[1].astype(jnp.int32)
    src2 = src * 2
    nfeat2 = nfeat.reshape(2 * N_NODES, HALF)
    agg = _sc_aggregate(nfeat2, efeat, src2, dst)
    return _mlp(agg, nfeat, W1, b1, W2, b2, gn1_alpha, gn1_gamma, gn1_beta,
                gn2_alpha, gn2_gamma, gn2_beta)


# speedup vs baseline: 4.8977x; 1.0194x over previous
import functools

import jax
import jax.numpy as jnp
from jax import lax
from jax.experimental import pallas as pl
from jax.experimental.pallas import tpu as pltpu
from jax.experimental.pallas import tpu_sc as plsc

N_NODES = 10000
N_EDGES = 160000
D = 256
HALF = 128
NSUB = 16
EDGES_PER_SUB = N_EDGES // NSUB
CHUNK = 80
NCHUNK = EDGES_PER_SUB // CHUNK
NODE_CHUNK = 80
N_NODE_CHUNKS = N_NODES // NODE_CHUNK
NODE_ROUNDS = (N_NODE_CHUNKS + NSUB - 1) // NSUB
GN_EPS = 1e-5


def _sc_aggregate(nfeat2, efeat, src2, dst):
    mesh = plsc.VectorSubcoreMesh(core_axis_name="core", subcore_axis_name="subcore")

    @functools.partial(
        pl.kernel,
        out_type=jax.ShapeDtypeStruct((N_NODES, D), jnp.float32),
        mesh=mesh,
        scratch_types=[
            pltpu.VMEM((2, CHUNK), jnp.int32),
            pltpu.VMEM((3, CHUNK), jnp.int32),
            pltpu.VMEM((2, CHUNK, HALF), jnp.float32),
            pltpu.VMEM((2, CHUNK, HALF), jnp.float32),
            pltpu.VMEM_SHARED((N_NODES, HALF), jnp.float32),
            pltpu.SemaphoreType.DMA((2,)),
            pltpu.SemaphoreType.DMA((2,)),
            pltpu.SemaphoreType.DMA((2,)),
            pltpu.SemaphoreType.DMA((2,)),
            pltpu.SemaphoreType.DMA((2,)),
            pltpu.SemaphoreType.DMA((2,)),
        ],
    )
    def k(nfeat2_hbm, efeat_hbm, src2_hbm, dst_hbm, out_hbm,
          gidx_v, didx_v, grow_v, erow_v, agg_sh,
          semI, semD, semGn, semGe, semSg, semSe):
        c = lax.axis_index("core")
        s = lax.axis_index("subcore")
        f0 = c * HALF

        def e_window(e0):
            return efeat_hbm.at[pl.ds(e0, CHUNK), pl.ds(f0, HALF)]

        def start_idx(j, p, q):
            e0 = s * EDGES_PER_SUB + j * CHUNK
            pltpu.async_copy(src2_hbm.at[pl.ds(e0, CHUNK)], gidx_v.at[p], semI.at[p])
            pltpu.async_copy(dst_hbm.at[pl.ds(e0, CHUNK)], didx_v.at[q], semD.at[p])

        def wait_idx(j, p, q):
            e0 = s * EDGES_PER_SUB + j * CHUNK
            pltpu.make_async_copy(
                src2_hbm.at[pl.ds(e0, CHUNK)], gidx_v.at[p], semI.at[p]).wait()
            pltpu.make_async_copy(
                dst_hbm.at[pl.ds(e0, CHUNK)], didx_v.at[q], semD.at[p]).wait()

        def start_gather(j, p):
            e0 = s * EDGES_PER_SUB + j * CHUNK
            pltpu.async_copy(nfeat2_hbm.at[gidx_v.at[p]], grow_v.at[p], semGn.at[p])
            pltpu.async_copy(e_window(e0), erow_v.at[p], semGe.at[p])

        def wait_gather(p):
            pltpu.make_async_copy(
                nfeat2_hbm.at[gidx_v.at[p]], grow_v.at[p], semGn.at[p]).wait()
            pltpu.make_async_copy(e_window(0), erow_v.at[p], semGe.at[p]).wait()

        def start_scatter(p, q):
            pltpu.async_copy(grow_v.at[p], agg_sh.at[didx_v.at[q]], semSg.at[p],
                             add=True)
            pltpu.async_copy(erow_v.at[p], agg_sh.at[didx_v.at[q]], semSe.at[p],
                             add=True)

        def wait_scatter(p, q):
            pltpu.make_async_copy(
                grow_v.at[p], agg_sh.at[didx_v.at[q]], semSg.at[p]).wait()
            pltpu.make_async_copy(
                erow_v.at[p], agg_sh.at[didx_v.at[q]], semSe.at[p]).wait()

        zeros16 = jnp.zeros((16,), jnp.float32)

        @pl.loop(0, NODE_CHUNK)
        def _(r):
            for i in range(HALF // 16):
                grow_v[0, r, pl.ds(16 * i, 16)] = zeros16

        @pl.loop(0, NODE_ROUNDS)
        def _(kk):
            cid = s + kk * NSUB

            @pl.when(cid < N_NODE_CHUNKS)
            def _():
                pltpu.async_copy(grow_v.at[0],
                                 agg_sh.at[pl.ds(cid * NODE_CHUNK, NODE_CHUNK)],
                                 semSg.at[0])

        @pl.loop(0, NODE_ROUNDS)
        def _(kk):
            @pl.when(s + kk * NSUB < N_NODE_CHUNKS)
            def _():
                pltpu.make_async_copy(
                    grow_v.at[0], agg_sh.at[pl.ds(0, NODE_CHUNK)],
                    semSg.at[0]).wait()

        plsc.subcore_barrier()

        start_idx(0, 0, 0)

        @pl.loop(0, NCHUNK)
        def _(j):
            p = lax.rem(j, 2)
            pn = 1 - p
            q = lax.rem(j, 3)

            @pl.when(j >= 1)
            def _():
                qm = lax.rem(j + 2, 3)
                wait_gather(pn)
                start_scatter(pn, qm)

            @pl.when(j >= 2)
            def _():
                wait_scatter(p, lax.rem(j + 1, 3))

            @pl.when(j + 1 < NCHUNK)
            def _():
                start_idx(j + 1, pn, lax.rem(j + 1, 3))

            wait_idx(j, p, q)
            for i in range(CHUNK // 16):
                gidx_v[p, pl.ds(16 * i, 16)] = gidx_v[p, pl.ds(16 * i, 16)] + c
            start_gather(j, p)

        pl_ = (NCHUNK - 1) % 2
        ql_ = (NCHUNK - 1) % 3
        wait_gather(pl_)
        start_scatter(pl_, ql_)
        wait_scatter(1 - pl_, (NCHUNK - 2) % 3)
        wait_scatter(pl_, ql_)

        plsc.subcore_barrier()

        @pl.loop(0, NODE_ROUNDS)
        def _(kk):
            cid = s + kk * NSUB

            @pl.when(cid < N_NODE_CHUNKS)
            def _():
                n0 = cid * NODE_CHUNK
                pltpu.async_copy(
                    agg_sh.at[pl.ds(n0, NODE_CHUNK)],
                    out_hbm.at[pl.ds(n0, NODE_CHUNK), pl.ds(f0, HALF)],
                    semSe.at[0])

        @pl.loop(0, NODE_ROUNDS)
        def _(kk):
            @pl.when(s + kk * NSUB < N_NODE_CHUNKS)
            def _():
                pltpu.make_async_copy(
                    agg_sh.at[pl.ds(0, NODE_CHUNK)],
                    out_hbm.at[pl.ds(0, NODE_CHUNK), pl.ds(f0, HALF)],
                    semSe.at[0]).wait()

    return k(nfeat2, efeat, src2, dst)


BR = 2000
NB = N_NODES // BR


def _norm_from_stats(st_ref, a):
    mean = st_ref[0:1, :] * (1.0 / N_NODES)
    msq = st_ref[1:2, :] * (1.0 / N_NODES)
    var = msq - (2.0 * a - a * a) * mean * mean
    rstd = lax.rsqrt(var + GN_EPS)
    return mean, rstd


def _fused_mlp_body(agg_ref, nfeat_ref, w1_ref, b1_ref, w2_ref, b2_ref,
                    a1_ref, g1_ref, be1_ref, a2_ref, g2_ref, be2_ref,
                    out_ref, rst1_s, rst2_s, st1_s, st2_s):
    p = pl.program_id(0)
    i = pl.program_id(1)
    rows = pl.ds(i * BR, BR)

    @pl.when(p == 0)
    def _():
        h = agg_ref[...] + nfeat_ref[...]
        y = jnp.dot(h, w1_ref[...], preferred_element_type=jnp.float32)
        y = y + b1_ref[...]
        rst1_s[rows, :] = y

        @pl.when(i == 0)
        def _():
            st1_s[...] = jnp.zeros_like(st1_s)

        st1_s[0:1, :] += jnp.sum(y, axis=0, keepdims=True)
        st1_s[1:2, :] += jnp.sum(y * y, axis=0, keepdims=True)

    @pl.when(p == 1)
    def _():
        a = a1_ref[...]
        mean, rstd = _norm_from_stats(st1_s, a)
        xn = g1_ref[...] * ((rst1_s[rows, :] - a * mean) * rstd) + be1_ref[...]
        r = jnp.maximum(xn, 0.0)
        y = jnp.dot(r, w2_ref[...], preferred_element_type=jnp.float32)
        y = y + b2_ref[...]
        rst2_s[rows, :] = y

        @pl.when(i == 0)
        def _():
            st2_s[...] = jnp.zeros_like(st2_s)

        st2_s[0:1, :] += jnp.sum(y, axis=0, keepdims=True)
        st2_s[1:2, :] += jnp.sum(y * y, axis=0, keepdims=True)

    @pl.when(p == 2)
    def _():
        a = a2_ref[...]
        mean, rstd = _norm_from_stats(st2_s, a)
        out_ref[...] = (
            g2_ref[...] * ((rst2_s[rows, :] - a * mean) * rstd) + be2_ref[...])


def _row(v):
    return v.reshape(1, -1)


def _mlp(agg, nfeat, W1, b1, W2, b2, gn1_alpha, gn1_gamma, gn1_beta,
         gn2_alpha, gn2_gamma, gn2_beta):
    D2 = 2 * D
    const = pl.BlockSpec((1, D2), lambda p, i: (0, 0))
    constD = pl.BlockSpec((1, D), lambda p, i: (0, 0))
    out = pl.pallas_call(
        _fused_mlp_body,
        grid=(3, NB),
        in_specs=[
            pl.BlockSpec((BR, D), lambda p, i: (i, 0)),
            pl.BlockSpec((BR, D), lambda p, i: (i, 0)),
            pl.BlockSpec((D, D2), lambda p, i: (0, 0)),
            const,
            pl.BlockSpec((D2, D), lambda p, i: (0, 0)),
            constD,
            const, const, const,
            constD, constD, constD,
        ],
        out_specs=pl.BlockSpec((BR, D), lambda p, i: (i, 0)),
        out_shape=jax.ShapeDtypeStruct((N_NODES, D), jnp.float32),
        scratch_shapes=[
            pltpu.VMEM((N_NODES, D2), jnp.float32),
            pltpu.VMEM((N_NODES, D), jnp.float32),
            pltpu.VMEM((8, D2), jnp.float32),
            pltpu.VMEM((8, D), jnp.float32),
        ],
    )(agg, nfeat, W1, _row(b1), W2, _row(b2),
      _row(gn1_alpha), _row(gn1_gamma), _row(gn1_beta),
      _row(gn2_alpha), _row(gn2_gamma), _row(gn2_beta))
    return out


@jax.jit
def kernel(nfeat, efeat, edge_index, W1, b1, W2, b2,
           gn1_alpha, gn1_gamma, gn1_beta, gn2_alpha, gn2_gamma, gn2_beta):
    src = edge_index[0].astype(jnp.int32)
    dst = edge_index[1].astype(jnp.int32)
    src2 = src * 2
    nfeat2 = nfeat.reshape(2 * N_NODES, HALF)
    agg = _sc_aggregate(nfeat2, efeat, src2, dst)
    return _mlp(agg, nfeat, W1, b1, W2, b2, gn1_alpha, gn1_gamma, gn1_beta,
                gn2_alpha, gn2_gamma, gn2_beta)


# speedup vs baseline: 5.0910x; 1.0395x over previous
import functools

import jax
import jax.numpy as jnp
from jax import lax
from jax.experimental import pallas as pl
from jax.experimental.pallas import tpu as pltpu
from jax.experimental.pallas import tpu_sc as plsc

N_NODES = 10000
N_EDGES = 160000
D = 256
HALF = 128
NSUB = 16
EDGES_PER_SUB = N_EDGES // NSUB
CHUNK = 80
NCHUNK = EDGES_PER_SUB // CHUNK
NODE_CHUNK = 80
N_NODE_CHUNKS = N_NODES // NODE_CHUNK
NODE_ROUNDS = (N_NODE_CHUNKS + NSUB - 1) // NSUB
GN_EPS = 1e-5


def _sc_aggregate(nfeat2, efeat, src2, dst):
    mesh = plsc.VectorSubcoreMesh(core_axis_name="core", subcore_axis_name="subcore")

    @functools.partial(
        pl.kernel,
        out_type=jax.ShapeDtypeStruct((N_NODES, D), jnp.float32),
        mesh=mesh,
        scratch_types=[
            pltpu.VMEM((2, CHUNK), jnp.int32),
            pltpu.VMEM((3, CHUNK), jnp.int32),
            pltpu.VMEM((2, CHUNK, HALF), jnp.float32),
            pltpu.VMEM((2, CHUNK, HALF), jnp.float32),
            pltpu.VMEM_SHARED((N_NODES, HALF), jnp.float32),
            pltpu.SemaphoreType.DMA((2,)),
            pltpu.SemaphoreType.DMA((2,)),
            pltpu.SemaphoreType.DMA((2,)),
            pltpu.SemaphoreType.DMA((2,)),
            pltpu.SemaphoreType.DMA((2,)),
            pltpu.SemaphoreType.DMA((2,)),
        ],
    )
    def k(nfeat2_hbm, efeat_hbm, src2_hbm, dst_hbm, out_hbm,
          gidx_v, didx_v, grow_v, erow_v, agg_sh,
          semI, semD, semGn, semGe, semSg, semSe):
        c = lax.axis_index("core")
        s = lax.axis_index("subcore")
        f0 = c * HALF

        def e_window(e0):
            return efeat_hbm.at[pl.ds(e0, CHUNK), pl.ds(f0, HALF)]

        def start_idx(j, p, q):
            e0 = s * EDGES_PER_SUB + j * CHUNK
            pltpu.async_copy(src2_hbm.at[pl.ds(e0, CHUNK)], gidx_v.at[p], semI.at[p])
            pltpu.async_copy(dst_hbm.at[pl.ds(e0, CHUNK)], didx_v.at[q], semD.at[p])

        def wait_idx(j, p, q):
            e0 = s * EDGES_PER_SUB + j * CHUNK
            pltpu.make_async_copy(
                src2_hbm.at[pl.ds(e0, CHUNK)], gidx_v.at[p], semI.at[p]).wait()
            pltpu.make_async_copy(
                dst_hbm.at[pl.ds(e0, CHUNK)], didx_v.at[q], semD.at[p]).wait()

        def start_gather(j, p):
            e0 = s * EDGES_PER_SUB + j * CHUNK
            pltpu.async_copy(nfeat2_hbm.at[gidx_v.at[p]], grow_v.at[p], semGn.at[p])
            pltpu.async_copy(e_window(e0), erow_v.at[p], semGe.at[p])

        def wait_gather(p):
            pltpu.make_async_copy(
                nfeat2_hbm.at[gidx_v.at[p]], grow_v.at[p], semGn.at[p]).wait()
            pltpu.make_async_copy(e_window(0), erow_v.at[p], semGe.at[p]).wait()

        def start_scatter(p, q):
            pltpu.async_copy(grow_v.at[p], agg_sh.at[didx_v.at[q]], semSg.at[p],
                             add=True)
            pltpu.async_copy(erow_v.at[p], agg_sh.at[didx_v.at[q]], semSe.at[p],
                             add=True)

        def wait_scatter(p, q):
            pltpu.make_async_copy(
                grow_v.at[p], agg_sh.at[didx_v.at[q]], semSg.at[p]).wait()
            pltpu.make_async_copy(
                erow_v.at[p], agg_sh.at[didx_v.at[q]], semSe.at[p]).wait()

        zeros16 = jnp.zeros((16,), jnp.float32)

        @pl.loop(0, NODE_CHUNK)
        def _(r):
            for i in range(HALF // 16):
                grow_v[0, r, pl.ds(16 * i, 16)] = zeros16

        @pl.loop(0, NODE_ROUNDS)
        def _(kk):
            cid = s + kk * NSUB

            @pl.when(cid < N_NODE_CHUNKS)
            def _():
                pltpu.async_copy(grow_v.at[0],
                                 agg_sh.at[pl.ds(cid * NODE_CHUNK, NODE_CHUNK)],
                                 semSg.at[0])

        @pl.loop(0, NODE_ROUNDS)
        def _(kk):
            @pl.when(s + kk * NSUB < N_NODE_CHUNKS)
            def _():
                pltpu.make_async_copy(
                    grow_v.at[0], agg_sh.at[pl.ds(0, NODE_CHUNK)],
                    semSg.at[0]).wait()

        plsc.subcore_barrier()

        start_idx(0, 0, 0)

        @pl.loop(0, NCHUNK)
        def _(j):
            p = lax.rem(j, 2)
            pn = 1 - p
            q = lax.rem(j, 3)

            @pl.when(j >= 1)
            def _():
                qm = lax.rem(j + 2, 3)
                wait_gather(pn)
                start_scatter(pn, qm)

            @pl.when(j >= 2)
            def _():
                wait_scatter(p, lax.rem(j + 1, 3))

            @pl.when(j + 1 < NCHUNK)
            def _():
                start_idx(j + 1, pn, lax.rem(j + 1, 3))

            wait_idx(j, p, q)
            for i in range(CHUNK // 16):
                gidx_v[p, pl.ds(16 * i, 16)] = gidx_v[p, pl.ds(16 * i, 16)] + c
            start_gather(j, p)

        pl_ = (NCHUNK - 1) % 2
        ql_ = (NCHUNK - 1) % 3
        wait_gather(pl_)
        start_scatter(pl_, ql_)
        wait_scatter(1 - pl_, (NCHUNK - 2) % 3)
        wait_scatter(pl_, ql_)

        plsc.subcore_barrier()

        @pl.loop(0, NODE_ROUNDS)
        def _(kk):
            cid = s + kk * NSUB

            @pl.when(cid < N_NODE_CHUNKS)
            def _():
                n0 = cid * NODE_CHUNK
                pltpu.async_copy(
                    agg_sh.at[pl.ds(n0, NODE_CHUNK)],
                    out_hbm.at[pl.ds(n0, NODE_CHUNK), pl.ds(f0, HALF)],
                    semSe.at[0])

        @pl.loop(0, NODE_ROUNDS)
        def _(kk):
            @pl.when(s + kk * NSUB < N_NODE_CHUNKS)
            def _():
                pltpu.make_async_copy(
                    agg_sh.at[pl.ds(0, NODE_CHUNK)],
                    out_hbm.at[pl.ds(0, NODE_CHUNK), pl.ds(f0, HALF)],
                    semSe.at[0]).wait()

    return k(nfeat2, efeat, src2, dst)


BR = 2000
NB = N_NODES // BR


def _norm_from_stats(st_ref, a):
    mean = st_ref[0:1, :] * (1.0 / N_NODES)
    msq = st_ref[1:2, :] * (1.0 / N_NODES)
    var = msq - (2.0 * a - a * a) * mean * mean
    rstd = lax.rsqrt(var + GN_EPS)
    return mean, rstd


def _fused_mlp_body(agg_ref, nfeat_ref, w1_ref, b1_ref, w2_ref, b2_ref,
                    a1_ref, g1_ref, be1_ref, a2_ref, g2_ref, be2_ref,
                    out_ref, rst1_s, rst2_s, st1_s, st2_s):
    p = pl.program_id(0)
    i = pl.program_id(1)
    rows = pl.ds(i * BR, BR)

    @pl.when(p == 0)
    def _():
        h = (agg_ref[...] + nfeat_ref[...]).astype(jnp.bfloat16)
        y = jnp.dot(h, w1_ref[...], preferred_element_type=jnp.float32)
        y = y + b1_ref[...]
        rst1_s[rows, :] = y

        @pl.when(i == 0)
        def _():
            st1_s[...] = jnp.zeros_like(st1_s)

        st1_s[0:1, :] += jnp.sum(y, axis=0, keepdims=True)
        st1_s[1:2, :] += jnp.sum(y * y, axis=0, keepdims=True)

    @pl.when(p == 1)
    def _():
        a = a1_ref[...]
        mean, rstd = _norm_from_stats(st1_s, a)
        xn = g1_ref[...] * ((rst1_s[rows, :] - a * mean) * rstd) + be1_ref[...]
        r = jnp.maximum(xn, 0.0).astype(jnp.bfloat16)
        y = jnp.dot(r, w2_ref[...], preferred_element_type=jnp.float32)
        y = y + b2_ref[...]
        rst2_s[rows, :] = y

        @pl.when(i == 0)
        def _():
            st2_s[...] = jnp.zeros_like(st2_s)

        st2_s[0:1, :] += jnp.sum(y, axis=0, keepdims=True)
        st2_s[1:2, :] += jnp.sum(y * y, axis=0, keepdims=True)

    @pl.when(p == 2)
    def _():
        a = a2_ref[...]
        mean, rstd = _norm_from_stats(st2_s, a)
        out_ref[...] = (
            g2_ref[...] * ((rst2_s[rows, :] - a * mean) * rstd) + be2_ref[...])


def _row(v):
    return v.reshape(1, -1)


def _mlp(agg, nfeat, W1, b1, W2, b2, gn1_alpha, gn1_gamma, gn1_beta,
         gn2_alpha, gn2_gamma, gn2_beta):
    D2 = 2 * D
    const = pl.BlockSpec((1, D2), lambda p, i: (0, 0))
    constD = pl.BlockSpec((1, D), lambda p, i: (0, 0))
    out = pl.pallas_call(
        _fused_mlp_body,
        grid=(3, NB),
        in_specs=[
            pl.BlockSpec((BR, D), lambda p, i: (jnp.where(p == 0, i, 0), 0)),
            pl.BlockSpec((BR, D), lambda p, i: (jnp.where(p == 0, i, 0), 0)),
            pl.BlockSpec((D, D2), lambda p, i: (0, 0)),
            const,
            pl.BlockSpec((D2, D), lambda p, i: (0, 0)),
            constD,
            const, const, const,
            constD, constD, constD,
        ],
        out_specs=pl.BlockSpec((BR, D), lambda p, i: (jnp.where(p == 2, i, 0), 0)),
        out_shape=jax.ShapeDtypeStruct((N_NODES, D), jnp.float32),
        scratch_shapes=[
            pltpu.VMEM((N_NODES, D2), jnp.float32),
            pltpu.VMEM((N_NODES, D), jnp.float32),
            pltpu.VMEM((8, D2), jnp.float32),
            pltpu.VMEM((8, D), jnp.float32),
        ],
    )(agg, nfeat, W1.astype(jnp.bfloat16), _row(b1), W2.astype(jnp.bfloat16),
      _row(b2),
      _row(gn1_alpha), _row(gn1_gamma), _row(gn1_beta),
      _row(gn2_alpha), _row(gn2_gamma), _row(gn2_beta))
    return out


@jax.jit
def kernel(nfeat, efeat, edge_index, W1, b1, W2, b2,
           gn1_alpha, gn1_gamma, gn1_beta, gn2_alpha, gn2_gamma, gn2_beta):
    src = edge_index[0].astype(jnp.int32)
    dst = edge_index[1].astype(jnp.int32)
    src2 = src * 2
    nfeat2 = nfeat.reshape(2 * N_NODES, HALF)
    agg = _sc_aggregate(nfeat2, efeat, src2, dst)
    return _mlp(agg, nfeat, W1, b1, W2, b2, gn1_alpha, gn1_gamma, gn1_beta,
                gn2_alpha, gn2_gamma, gn2_beta)


# speedup vs baseline: 5.2261x; 1.0265x over previous
import functools

import jax
import jax.numpy as jnp
from jax import lax
from jax.experimental import pallas as pl
from jax.experimental.pallas import tpu as pltpu
from jax.experimental.pallas import tpu_sc as plsc

N_NODES = 10000
N_EDGES = 160000
D = 256
HALF = 128
NSUB = 16
EDGES_PER_SUB = N_EDGES // NSUB
CHUNK = 80
NCHUNK = EDGES_PER_SUB // CHUNK
NODE_CHUNK = 80
N_NODE_CHUNKS = N_NODES // NODE_CHUNK
NODE_ROUNDS = (N_NODE_CHUNKS + NSUB - 1) // NSUB
GN_EPS = 1e-5


def _sc_aggregate(nfeat2, efeat, ei_flat):
    mesh = plsc.VectorSubcoreMesh(core_axis_name="core", subcore_axis_name="subcore")

    @functools.partial(
        pl.kernel,
        out_type=jax.ShapeDtypeStruct((N_NODES, D), jnp.float32),
        mesh=mesh,
        scratch_types=[
            pltpu.VMEM((2, CHUNK), jnp.int32),
            pltpu.VMEM((3, CHUNK), jnp.int32),
            pltpu.VMEM((2, CHUNK, HALF), jnp.float32),
            pltpu.VMEM((2, CHUNK, HALF), jnp.float32),
            pltpu.VMEM_SHARED((N_NODES, HALF), jnp.float32),
            pltpu.SemaphoreType.DMA((2,)),
            pltpu.SemaphoreType.DMA((2,)),
            pltpu.SemaphoreType.DMA((2,)),
            pltpu.SemaphoreType.DMA((2,)),
            pltpu.SemaphoreType.DMA((2,)),
            pltpu.SemaphoreType.DMA((2,)),
        ],
    )
    def k(nfeat2_hbm, efeat_hbm, ei_hbm, out_hbm,
          gidx_v, didx_v, grow_v, erow_v, agg_sh,
          semI, semD, semGn, semGe, semSg, semSe):
        c = lax.axis_index("core")
        s = lax.axis_index("subcore")
        f0 = c * HALF

        def e_window(e0):
            return efeat_hbm.at[pl.ds(e0, CHUNK), pl.ds(f0, HALF)]

        def start_idx(j, p, q):
            e0 = s * EDGES_PER_SUB + j * CHUNK
            pltpu.async_copy(ei_hbm.at[pl.ds(e0, CHUNK)], gidx_v.at[p], semI.at[p])
            pltpu.async_copy(ei_hbm.at[pl.ds(N_EDGES + e0, CHUNK)], didx_v.at[q],
                             semD.at[p])

        def wait_idx(j, p, q):
            e0 = s * EDGES_PER_SUB + j * CHUNK
            pltpu.make_async_copy(
                ei_hbm.at[pl.ds(e0, CHUNK)], gidx_v.at[p], semI.at[p]).wait()
            pltpu.make_async_copy(
                ei_hbm.at[pl.ds(N_EDGES + e0, CHUNK)], didx_v.at[q],
                semD.at[p]).wait()

        def start_gather(j, p):
            e0 = s * EDGES_PER_SUB + j * CHUNK
            pltpu.async_copy(nfeat2_hbm.at[gidx_v.at[p]], grow_v.at[p], semGn.at[p])
            pltpu.async_copy(e_window(e0), erow_v.at[p], semGe.at[p])

        def wait_gather(p):
            pltpu.make_async_copy(
                nfeat2_hbm.at[gidx_v.at[p]], grow_v.at[p], semGn.at[p]).wait()
            pltpu.make_async_copy(e_window(0), erow_v.at[p], semGe.at[p]).wait()

        def start_scatter(p, q):
            pltpu.async_copy(grow_v.at[p], agg_sh.at[didx_v.at[q]], semSg.at[p],
                             add=True)
            pltpu.async_copy(erow_v.at[p], agg_sh.at[didx_v.at[q]], semSe.at[p],
                             add=True)

        def wait_scatter(p, q):
            pltpu.make_async_copy(
                grow_v.at[p], agg_sh.at[didx_v.at[q]], semSg.at[p]).wait()
            pltpu.make_async_copy(
                erow_v.at[p], agg_sh.at[didx_v.at[q]], semSe.at[p]).wait()

        zeros16 = jnp.zeros((16,), jnp.float32)

        @pl.loop(0, NODE_CHUNK)
        def _(r):
            for i in range(HALF // 16):
                grow_v[0, r, pl.ds(16 * i, 16)] = zeros16

        @pl.loop(0, NODE_ROUNDS)
        def _(kk):
            cid = s + kk * NSUB

            @pl.when(cid < N_NODE_CHUNKS)
            def _():
                pltpu.async_copy(grow_v.at[0],
                                 agg_sh.at[pl.ds(cid * NODE_CHUNK, NODE_CHUNK)],
                                 semSg.at[0])

        @pl.loop(0, NODE_ROUNDS)
        def _(kk):
            @pl.when(s + kk * NSUB < N_NODE_CHUNKS)
            def _():
                pltpu.make_async_copy(
                    grow_v.at[0], agg_sh.at[pl.ds(0, NODE_CHUNK)],
                    semSg.at[0]).wait()

        plsc.subcore_barrier()

        start_idx(0, 0, 0)

        @pl.loop(0, NCHUNK)
        def _(j):
            p = lax.rem(j, 2)
            pn = 1 - p
            q = lax.rem(j, 3)

            @pl.when(j >= 1)
            def _():
                qm = lax.rem(j + 2, 3)
                wait_gather(pn)
                start_scatter(pn, qm)

            @pl.when(j >= 2)
            def _():
                wait_scatter(p, lax.rem(j + 1, 3))

            @pl.when(j + 1 < NCHUNK)
            def _():
                start_idx(j + 1, pn, lax.rem(j + 1, 3))

            wait_idx(j, p, q)
            for i in range(CHUNK // 16):
                sl = pl.ds(16 * i, 16)
                gidx_v[p, sl] = gidx_v[p, sl] + gidx_v[p, sl] + c
            start_gather(j, p)

        pl_ = (NCHUNK - 1) % 2
        ql_ = (NCHUNK - 1) % 3
        wait_gather(pl_)
        start_scatter(pl_, ql_)
        wait_scatter(1 - pl_, (NCHUNK - 2) % 3)
        wait_scatter(pl_, ql_)

        plsc.subcore_barrier()

        @pl.loop(0, NODE_ROUNDS)
        def _(kk):
            cid = s + kk * NSUB

            @pl.when(cid < N_NODE_CHUNKS)
            def _():
                n0 = cid * NODE_CHUNK
                pltpu.async_copy(
                    agg_sh.at[pl.ds(n0, NODE_CHUNK)],
                    out_hbm.at[pl.ds(n0, NODE_CHUNK), pl.ds(f0, HALF)],
                    semSe.at[0])

        @pl.loop(0, NODE_ROUNDS)
        def _(kk):
            @pl.when(s + kk * NSUB < N_NODE_CHUNKS)
            def _():
                pltpu.make_async_copy(
                    agg_sh.at[pl.ds(0, NODE_CHUNK)],
                    out_hbm.at[pl.ds(0, NODE_CHUNK), pl.ds(f0, HALF)],
                    semSe.at[0]).wait()

    return k(nfeat2, efeat, ei_flat)


BR = 2000
NB = N_NODES // BR


def _norm_from_stats(st_ref, a):
    mean = st_ref[0:1, :] * (1.0 / N_NODES)
    msq = st_ref[1:2, :] * (1.0 / N_NODES)
    var = msq - (2.0 * a - a * a) * mean * mean
    rstd = lax.rsqrt(var + GN_EPS)
    return mean, rstd


def _fused_mlp_body(agg_ref, nfeat_ref, w1_ref, b1_ref, w2_ref, b2_ref,
                    a1_ref, g1_ref, be1_ref, a2_ref, g2_ref, be2_ref,
                    out_ref, rst1_s, rst2_s, st1_s, st2_s):
    p = pl.program_id(0)
    i = pl.program_id(1)
    rows = pl.ds(i * BR, BR)

    @pl.when(p == 0)
    def _():
        h = (agg_ref[...] + nfeat_ref[...]).astype(jnp.bfloat16)
        y = jnp.dot(h, w1_ref[...], preferred_element_type=jnp.float32)
        y = y + b1_ref[...]
        rst1_s[rows, :] = y

        @pl.when(i == 0)
        def _():
            st1_s[...] = jnp.zeros_like(st1_s)

        st1_s[0:1, :] += jnp.sum(y, axis=0, keepdims=True)
        st1_s[1:2, :] += jnp.sum(y * y, axis=0, keepdims=True)

    @pl.when(p == 1)
    def _():
        a = a1_ref[...]
        mean, rstd = _norm_from_stats(st1_s, a)
        xn = g1_ref[...] * ((rst1_s[rows, :] - a * mean) * rstd) + be1_ref[...]
        r = jnp.maximum(xn, 0.0).astype(jnp.bfloat16)
        y = jnp.dot(r, w2_ref[...], preferred_element_type=jnp.float32)
        y = y + b2_ref[...]
        rst2_s[rows, :] = y

        @pl.when(i == 0)
        def _():
            st2_s[...] = jnp.zeros_like(st2_s)

        st2_s[0:1, :] += jnp.sum(y, axis=0, keepdims=True)
        st2_s[1:2, :] += jnp.sum(y * y, axis=0, keepdims=True)

    @pl.when(p == 2)
    def _():
        a = a2_ref[...]
        mean, rstd = _norm_from_stats(st2_s, a)
        out_ref[...] = (
            g2_ref[...] * ((rst2_s[rows, :] - a * mean) * rstd) + be2_ref[...])


def _row(v):
    return v.reshape(1, -1)


def _mlp(agg, nfeat, W1, b1, W2, b2, gn1_alpha, gn1_gamma, gn1_beta,
         gn2_alpha, gn2_gamma, gn2_beta):
    D2 = 2 * D
    const = pl.BlockSpec((1, D2), lambda p, i: (0, 0))
    constD = pl.BlockSpec((1, D), lambda p, i: (0, 0))
    out = pl.pallas_call(
        _fused_mlp_body,
        grid=(3, NB),
        in_specs=[
            pl.BlockSpec((BR, D), lambda p, i: (jnp.where(p == 0, i, 0), 0)),
            pl.BlockSpec((BR, D), lambda p, i: (jnp.where(p == 0, i, 0), 0)),
            pl.BlockSpec((D, D2), lambda p, i: (0, 0)),
            const,
            pl.BlockSpec((D2, D), lambda p, i: (0, 0)),
            constD,
            const, const, const,
            constD, constD, constD,
        ],
        out_specs=pl.BlockSpec((BR, D), lambda p, i: (jnp.where(p == 2, i, 0), 0)),
        out_shape=jax.ShapeDtypeStruct((N_NODES, D), jnp.float32),
        scratch_shapes=[
            pltpu.VMEM((N_NODES, D2), jnp.float32),
            pltpu.VMEM((N_NODES, D), jnp.float32),
            pltpu.VMEM((8, D2), jnp.float32),
            pltpu.VMEM((8, D), jnp.float32),
        ],
    )(agg, nfeat, W1.astype(jnp.bfloat16), _row(b1), W2.astype(jnp.bfloat16),
      _row(b2),
      _row(gn1_alpha), _row(gn1_gamma), _row(gn1_beta),
      _row(gn2_alpha), _row(gn2_gamma), _row(gn2_beta))
    return out


@jax.jit
def kernel(nfeat, efeat, edge_index, W1, b1, W2, b2,
           gn1_alpha, gn1_gamma, gn1_beta, gn2_alpha, gn2_gamma, gn2_beta):
    ei_flat = edge_index.astype(jnp.int32).reshape(2 * N_EDGES)
    nfeat2 = nfeat.reshape(2 * N_NODES, HALF)
    agg = _sc_aggregate(nfeat2, efeat, ei_flat)
    return _mlp(agg, nfeat, W1, b1, W2, b2, gn1_alpha, gn1_gamma, gn1_beta,
                gn2_alpha, gn2_gamma, gn2_beta)


# speedup vs baseline: 6.3216x; 1.2096x over previous
import functools

import jax
import jax.numpy as jnp
from jax import lax
from jax.experimental import pallas as pl
from jax.experimental.pallas import tpu as pltpu
from jax.experimental.pallas import tpu_sc as plsc

N_NODES = 10000
N_EDGES = 160000
D = 256
HALF = 128
NSUB = 16
EDGES_PER_SUB = N_EDGES // NSUB
CHUNK = 56
NCHUNK = EDGES_PER_SUB // CHUNK
TAIL = EDGES_PER_SUB - NCHUNK * CHUNK
NDEEP = 3
IDEEP = 6
ZCHUNK = 40
NZ = N_NODES // ZCHUNK
Z_ROUNDS = (NZ + NSUB - 1) // NSUB
WCHUNK = 80
NW = N_NODES // WCHUNK
W_ROUNDS = (NW + NSUB - 1) // NSUB
GN_EPS = 1e-5


def _sc_aggregate(nfeat2, efeat, ei_flat):
    mesh = plsc.VectorSubcoreMesh(core_axis_name="core", subcore_axis_name="subcore")

    @functools.partial(
        pl.kernel,
        out_type=jax.ShapeDtypeStruct((N_NODES, D), jnp.float32),
        mesh=mesh,
        scratch_types=[
            pltpu.VMEM((IDEEP, 64), jnp.int32),
            pltpu.VMEM((IDEEP, CHUNK), jnp.int32),
            pltpu.VMEM((TAIL,), jnp.int32),
            pltpu.VMEM((TAIL,), jnp.int32),
            pltpu.VMEM((NDEEP, CHUNK, HALF), jnp.float32),
            pltpu.VMEM((NDEEP, CHUNK, HALF), jnp.float32),
            pltpu.VMEM_SHARED((N_NODES, HALF), jnp.float32),
            pltpu.SemaphoreType.DMA((IDEEP,)),
            pltpu.SemaphoreType.DMA((IDEEP,)),
            pltpu.SemaphoreType.DMA((NDEEP,)),
            pltpu.SemaphoreType.DMA((NDEEP,)),
            pltpu.SemaphoreType.DMA((NDEEP,)),
            pltpu.SemaphoreType.DMA((NDEEP,)),
        ],
    )
    def k(nfeat2_hbm, efeat_hbm, ei_hbm, out_hbm,
          gidx_v, didx_v, gidxT, didxT, grow_v, erow_v, agg_sh,
          semI, semD, semGn, semGe, semSg, semSe):
        c = lax.axis_index("core")
        s = lax.axis_index("subcore")
        f0 = c * HALF

        def e_window(e0):
            return efeat_hbm.at[pl.ds(e0, CHUNK), pl.ds(f0, HALF)]

        def gslice(u):
            return gidx_v.at[u, pl.ds(0, CHUNK)]

        def start_idx(j, u):
            e0 = s * EDGES_PER_SUB + j * CHUNK
            pltpu.async_copy(ei_hbm.at[pl.ds(e0, CHUNK)], gslice(u), semI.at[u])
            pltpu.async_copy(ei_hbm.at[pl.ds(N_EDGES + e0, CHUNK)], didx_v.at[u],
                             semD.at[u])

        def wait_idx(j, u):
            e0 = s * EDGES_PER_SUB + j * CHUNK
            pltpu.make_async_copy(
                ei_hbm.at[pl.ds(e0, CHUNK)], gslice(u), semI.at[u]).wait()
            pltpu.make_async_copy(
                ei_hbm.at[pl.ds(N_EDGES + e0, CHUNK)], didx_v.at[u],
                semD.at[u]).wait()

        def start_gather(j, m, u):
            e0 = s * EDGES_PER_SUB + j * CHUNK
            pltpu.async_copy(nfeat2_hbm.at[gslice(u)], grow_v.at[m], semGn.at[m])
            pltpu.async_copy(e_window(e0), erow_v.at[m], semGe.at[m])

        def wait_gather(m, u):
            pltpu.make_async_copy(
                nfeat2_hbm.at[gslice(u)], grow_v.at[m], semGn.at[m]).wait()
            pltpu.make_async_copy(e_window(0), erow_v.at[m], semGe.at[m]).wait()

        def start_scatter(m, u):
            pltpu.async_copy(grow_v.at[m], agg_sh.at[didx_v.at[u]], semSg.at[m],
                             add=True)
            pltpu.async_copy(erow_v.at[m], agg_sh.at[didx_v.at[u]], semSe.at[m],
                             add=True)

        def wait_scatter(m, u):
            pltpu.make_async_copy(
                grow_v.at[m], agg_sh.at[didx_v.at[u]], semSg.at[m]).wait()
            pltpu.make_async_copy(
                erow_v.at[m], agg_sh.at[didx_v.at[u]], semSe.at[m]).wait()

        def double_plus_c(slot):
            for i in range(64 // 16):
                sl = pl.ds(16 * i, 16)
                gidx_v[slot, sl] = gidx_v[slot, sl] + gidx_v[slot, sl] + c

        zeros16 = jnp.zeros((16,), jnp.float32)

        @pl.loop(0, ZCHUNK)
        def _(r):
            for i in range(HALF // 16):
                grow_v[0, r, pl.ds(16 * i, 16)] = zeros16

        @pl.loop(0, Z_ROUNDS)
        def _(kk):
            cid = s + kk * NSUB

            @pl.when(cid < NZ)
            def _():
                pltpu.async_copy(grow_v.at[0, pl.ds(0, ZCHUNK)],
                                 agg_sh.at[pl.ds(cid * ZCHUNK, ZCHUNK)],
                                 semSg.at[0])

        @pl.loop(0, Z_ROUNDS)
        def _(kk):
            @pl.when(s + kk * NSUB < NZ)
            def _():
                pltpu.make_async_copy(
                    grow_v.at[0, pl.ds(0, ZCHUNK)], agg_sh.at[pl.ds(0, ZCHUNK)],
                    semSg.at[0]).wait()

        plsc.subcore_barrier()

        start_idx(0, 0)
        start_idx(1, 1)

        @pl.loop(0, NCHUNK)
        def _(j):
            m = lax.rem(j, NDEEP)
            u = lax.rem(j, IDEEP)

            @pl.when(j >= 3)
            def _():
                wait_scatter(m, lax.rem(j + 3, IDEEP))

            wait_idx(j, u)
            double_plus_c(u)
            start_gather(j, m, u)

            @pl.when(j >= 2)
            def _():
                m2 = lax.rem(j + 1, NDEEP)
                u2 = lax.rem(j + 4, IDEEP)
                wait_gather(m2, u2)
                start_scatter(m2, u2)

            @pl.when(j + 2 < NCHUNK)
            def _():
                start_idx(j + 2, lax.rem(j + 2, IDEEP))

        for jj in (NCHUNK - 2, NCHUNK - 1):
            wait_gather(jj % NDEEP, jj % IDEEP)
            start_scatter(jj % NDEEP, jj % IDEEP)
        for jj in (NCHUNK - 3, NCHUNK - 2, NCHUNK - 1):
            wait_scatter(jj % NDEEP, jj % IDEEP)

        if True:
            tbase = s * EDGES_PER_SUB + NCHUNK * CHUNK
            pltpu.sync_copy(ei_hbm.at[pl.ds(tbase, TAIL)], gidxT)
            pltpu.sync_copy(ei_hbm.at[pl.ds(N_EDGES + tbase, TAIL)], didxT)
            for i in range(TAIL // 16):
                sl = pl.ds(16 * i, 16)
                gidxT[sl] = gidxT[sl] + gidxT[sl] + c
            pltpu.sync_copy(nfeat2_hbm.at[gidxT], grow_v.at[0, pl.ds(0, TAIL)])
            pltpu.sync_copy(efeat_hbm.at[pl.ds(tbase, TAIL), pl.ds(f0, HALF)],
                            erow_v.at[0, pl.ds(0, TAIL)])
            pltpu.sync_copy(grow_v.at[0, pl.ds(0, TAIL)], agg_sh.at[didxT],
                            add=True)
            pltpu.sync_copy(erow_v.at[0, pl.ds(0, TAIL)], agg_sh.at[didxT],
                            add=True)

        plsc.subcore_barrier()

        @pl.loop(0, W_ROUNDS)
        def _(kk):
            cid = s + kk * NSUB

            @pl.when(cid < NW)
            def _():
                n0 = cid * WCHUNK
                pltpu.async_copy(
                    agg_sh.at[pl.ds(n0, WCHUNK)],
                    out_hbm.at[pl.ds(n0, WCHUNK), pl.ds(f0, HALF)],
                    semSe.at[0])

        @pl.loop(0, W_ROUNDS)
        def _(kk):
            @pl.when(s + kk * NSUB < NW)
            def _():
                pltpu.make_async_copy(
                    agg_sh.at[pl.ds(0, WCHUNK)],
                    out_hbm.at[pl.ds(0, WCHUNK), pl.ds(f0, HALF)],
                    semSe.at[0]).wait()

    return k(nfeat2, efeat, ei_flat)


BR = 2000
NB = N_NODES // BR


def _norm_from_stats(st_ref, a):
    mean = st_ref[0:1, :] * (1.0 / N_NODES)
    msq = st_ref[1:2, :] * (1.0 / N_NODES)
    var = msq - (2.0 * a - a * a) * mean * mean
    rstd = lax.rsqrt(var + GN_EPS)
    return mean, rstd


def _fused_mlp_body(agg_ref, nfeat_ref, w1_ref, b1_ref, w2_ref, b2_ref,
                    a1_ref, g1_ref, be1_ref, a2_ref, g2_ref, be2_ref,
                    out_ref, rst1_s, rst2_s, st1_s, st2_s):
    p = pl.program_id(0)
    i = pl.program_id(1)
    rows = pl.ds(i * BR, BR)

    @pl.when(p == 0)
    def _():
        h = (agg_ref[...] + nfeat_ref[...]).astype(jnp.bfloat16)
        y = jnp.dot(h, w1_ref[...], preferred_element_type=jnp.float32)
        y = y + b1_ref[...]
        rst1_s[rows, :] = y

        @pl.when(i == 0)
        def _():
            st1_s[...] = jnp.zeros_like(st1_s)

        st1_s[0:1, :] += jnp.sum(y, axis=0, keepdims=True)
        st1_s[1:2, :] += jnp.sum(y * y, axis=0, keepdims=True)

    @pl.when(p == 1)
    def _():
        a = a1_ref[...]
        mean, rstd = _norm_from_stats(st1_s, a)
        xn = g1_ref[...] * ((rst1_s[rows, :] - a * mean) * rstd) + be1_ref[...]
        r = jnp.maximum(xn, 0.0).astype(jnp.bfloat16)
        y = jnp.dot(r, w2_ref[...], preferred_element_type=jnp.float32)
        y = y + b2_ref[...]
        rst2_s[rows, :] = y

        @pl.when(i == 0)
        def _():
            st2_s[...] = jnp.zeros_like(st2_s)

        st2_s[0:1, :] += jnp.sum(y, axis=0, keepdims=True)
        st2_s[1:2, :] += jnp.sum(y * y, axis=0, keepdims=True)

    @pl.when(p == 2)
    def _():
        a = a2_ref[...]
        mean, rstd = _norm_from_stats(st2_s, a)
        out_ref[...] = (
            g2_ref[...] * ((rst2_s[rows, :] - a * mean) * rstd) + be2_ref[...])


def _row(v):
    return v.reshape(1, -1)


def _mlp(agg, nfeat, W1, b1, W2, b2, gn1_alpha, gn1_gamma, gn1_beta,
         gn2_alpha, gn2_gamma, gn2_beta):
    D2 = 2 * D
    const = pl.BlockSpec((1, D2), lambda p, i: (0, 0))
    constD = pl.BlockSpec((1, D), lambda p, i: (0, 0))
    out = pl.pallas_call(
        _fused_mlp_body,
        grid=(3, NB),
        in_specs=[
            pl.BlockSpec((BR, D), lambda p, i: (jnp.where(p == 0, i, 0), 0)),
            pl.BlockSpec((BR, D), lambda p, i: (jnp.where(p == 0, i, 0), 0)),
            pl.BlockSpec((D, D2), lambda p, i: (0, 0)),
            const,
            pl.BlockSpec((D2, D), lambda p, i: (0, 0)),
            constD,
            const, const, const,
            constD, constD, constD,
        ],
        out_specs=pl.BlockSpec((BR, D), lambda p, i: (jnp.where(p == 2, i, 0), 0)),
        out_shape=jax.ShapeDtypeStruct((N_NODES, D), jnp.float32),
        scratch_shapes=[
            pltpu.VMEM((N_NODES, D2), jnp.float32),
            pltpu.VMEM((N_NODES, D), jnp.float32),
            pltpu.VMEM((8, D2), jnp.float32),
            pltpu.VMEM((8, D), jnp.float32),
        ],
    )(agg, nfeat, W1.astype(jnp.bfloat16), _row(b1), W2.astype(jnp.bfloat16),
      _row(b2),
      _row(gn1_alpha), _row(gn1_gamma), _row(gn1_beta),
      _row(gn2_alpha), _row(gn2_gamma), _row(gn2_beta))
    return out


@jax.jit
def kernel(nfeat, efeat, edge_index, W1, b1, W2, b2,
           gn1_alpha, gn1_gamma, gn1_beta, gn2_alpha, gn2_gamma, gn2_beta):
    ei_flat = edge_index.astype(jnp.int32).reshape(2 * N_EDGES)
    nfeat2 = nfeat.reshape(2 * N_NODES, HALF)
    agg = _sc_aggregate(nfeat2, efeat, ei_flat)
    return _mlp(agg, nfeat, W1, b1, W2, b2, gn1_alpha, gn1_gamma, gn1_beta,
                gn2_alpha, gn2_gamma, gn2_beta)
